# baseline (speedup 1.0000x reference)
import functools

import jax
import jax.numpy as jnp
from jax import lax
from jax.experimental import pallas as pl
from jax.experimental.pallas import tpu as pltpu
from jax.experimental.pallas import tpu_sc as plsc

B = 4096
SEQ = 200
DIM = 64
N = B * SEQ * DIM
L = 16
NC = 2
NS = 16
NW = NC * NS
BANDS = SEQ * (DIM // 8)
PW_BANDS = BANDS // NW
CHEL = 16384
NCH = PW_BANDS * 2
NBUF = 4
PD = 2
GROUPS = NCH // NBUF
PWP = PW_BANDS * 8 * L

_mesh = plsc.VectorSubcoreMesh(core_axis_name="c", subcore_axis_name="s")


@functools.partial(
    pl.kernel,
    out_type=jax.ShapeDtypeStruct((N,), jnp.float32),
    mesh=_mesh,
    scratch_types=dict(
        pos_v=pltpu.VMEM((PWP,), jnp.float32),
        bufs=[pltpu.VMEM((CHEL,), jnp.float32) for _ in range(NBUF)],
        isems=[pltpu.SemaphoreType.DMA for _ in range(NBUF)],
        osems=[pltpu.SemaphoreType.DMA for _ in range(NBUF)],
    ),
)
def _sc_add(x_hbm, pos_hbm, out_hbm, *, pos_v, bufs, isems, osems):
    wid = lax.axis_index("s") * NC + lax.axis_index("c")
    base = wid * (NCH * CHEL)

    pltpu.sync_copy(pos_hbm.at[pl.ds(wid * PWP, PWP)], pos_v)

    def compute_chunk(buf, c):
        pb = (c >> 1) * (8 * L)
        pv = [pos_v[pl.ds(pb + dr * L, L)] for dr in range(8)]

        def body(bq, carry):
            tb = pl.multiple_of(bq * 1024, 1024)
            for dr in range(8):
                for cc in range(8):
                    plsc.addupdate(
                        buf.at[pl.ds(tb + dr * 128 + cc * L, L)], pv[dr])
            return carry
        lax.fori_loop(0, CHEL // 1024, body, 0)

    def slot(c, k, *, osem_wait=True, prefetch=True):
        off = base + c * CHEL
        pltpu.make_async_copy(x_hbm.at[pl.ds(off, CHEL)], bufs[k], isems[k]).wait()
        compute_chunk(bufs[k], c)
        pltpu.async_copy(bufs[k], out_hbm.at[pl.ds(off, CHEL)], osems[k])
        if prefetch:
            kp = (k + PD) % NBUF
            if osem_wait:
                pltpu.make_async_copy(
                    bufs[kp], out_hbm.at[pl.ds(off + (PD - NBUF) * CHEL, CHEL)],
                    osems[kp]).wait()
            pltpu.async_copy(x_hbm.at[pl.ds(off + PD * CHEL, CHEL)],
                             bufs[kp], isems[kp])

    for k in range(PD):
        pltpu.async_copy(x_hbm.at[pl.ds(base + k * CHEL, CHEL)],
                         bufs[k], isems[k])

    for k in range(NBUF):
        slot(k, k, osem_wait=(k + PD - NBUF >= 0))

    def group_body(g, carry):
        for k in range(NBUF):
            slot(g * NBUF + k, k)
        return carry
    lax.fori_loop(1, GROUPS - 1, group_body, 0)

    for k in range(NBUF):
        slot((GROUPS - 1) * NBUF + k, k, prefetch=(k < NBUF - PD))

    for j in range(NBUF):
        c = NCH - NBUF + j
        pltpu.make_async_copy(bufs[c % NBUF],
                              out_hbm.at[pl.ds(base + c * CHEL, CHEL)],
                              osems[c % NBUF]).wait()


def kernel(x, pos_table):
    x1 = x.reshape(B // 128, 128, SEQ, DIM // 8, 8)
    x1 = x1.transpose(2, 3, 0, 4, 1).reshape(N)
    pos_rep = jnp.repeat(pos_table.reshape(-1), L,
                         total_repeat_length=SEQ * DIM * L)
    o1 = _sc_add(x1, pos_rep)
    o5 = o1.reshape(SEQ, DIM // 8, B // 128, 8, 128)
    return o5.transpose(2, 4, 0, 1, 3).reshape(B, SEQ, DIM)


# speedup vs baseline: 15.1179x; 15.1179x over previous
import functools

import jax
import jax.numpy as jnp
from jax import lax
from jax.experimental import pallas as pl
from jax.experimental.pallas import tpu as pltpu
from jax.experimental.pallas import tpu_sc as plsc

B = 4096
SEQ = 200
DIM = 64
N = B * SEQ * DIM
L = 16
NC = 2
NS = 16
NW = NC * NS
BANDS = SEQ * (DIM // 8)
PW_BANDS = BANDS // NW
CHEL = 16384
NCH = PW_BANDS * 2
NBUF = 4
PD = 2
GROUPS = NCH // NBUF
PWP = PW_BANDS * 8

_mesh = plsc.VectorSubcoreMesh(core_axis_name="c", subcore_axis_name="s")


@functools.partial(
    pl.kernel,
    out_type=jax.ShapeDtypeStruct((N,), jnp.float32),
    mesh=_mesh,
    compiler_params=pltpu.CompilerParams(needs_layout_passes=False),
    scratch_types=dict(
        pos_v=pltpu.VMEM((PWP,), jnp.float32),
        bufs=[pltpu.VMEM((CHEL,), jnp.float32) for _ in range(NBUF)],
        isems=[pltpu.SemaphoreType.DMA for _ in range(NBUF)],
        osems=[pltpu.SemaphoreType.DMA for _ in range(NBUF)],
    ),
)
def _sc_add(x_hbm, pos_hbm, out_hbm, *, pos_v, bufs, isems, osems):
    wid = lax.axis_index("s") * NC + lax.axis_index("c")
    base = wid * (NCH * CHEL)

    pltpu.sync_copy(pos_hbm.at[pl.ds(wid * PWP, PWP)], pos_v)

    def compute_chunk(buf, c):
        pb = (c >> 1) * 8
        pv = [plsc.load_gather(pos_v, [jnp.full((L,), pb + dr, jnp.int32)])
              for dr in range(8)]

        def body(bq, carry):
            tb = pl.multiple_of(bq * 1024, 1024)
            for dr in range(8):
                for cc in range(8):
                    plsc.addupdate(
                        buf.at[pl.ds(tb + dr * 128 + cc * L, L)], pv[dr])
            return carry
        lax.fori_loop(0, CHEL // 1024, body, 0)

    def slot(c, k, *, osem_wait=True, prefetch=True):
        off = base + c * CHEL
        pltpu.make_async_copy(x_hbm.at[pl.ds(off, CHEL)], bufs[k], isems[k]).wait()
        compute_chunk(bufs[k], c)
        pltpu.async_copy(bufs[k], out_hbm.at[pl.ds(off, CHEL)], osems[k])
        if prefetch:
            kp = (k + PD) % NBUF
            if osem_wait:
                pltpu.make_async_copy(
                    bufs[kp], out_hbm.at[pl.ds(off + (PD - NBUF) * CHEL, CHEL)],
                    osems[kp]).wait()
            pltpu.async_copy(x_hbm.at[pl.ds(off + PD * CHEL, CHEL)],
                             bufs[kp], isems[kp])

    for k in range(PD):
        pltpu.async_copy(x_hbm.at[pl.ds(base + k * CHEL, CHEL)],
                         bufs[k], isems[k])

    for k in range(NBUF):
        slot(k, k, osem_wait=(k + PD - NBUF >= 0))

    def group_body(g, carry):
        for k in range(NBUF):
            slot(g * NBUF + k, k)
        return carry
    lax.fori_loop(1, GROUPS - 1, group_body, 0)

    for k in range(NBUF):
        slot((GROUPS - 1) * NBUF + k, k, prefetch=(k < NBUF - PD))

    for j in range(NBUF):
        c = NCH - NBUF + j
        pltpu.make_async_copy(bufs[c % NBUF],
                              out_hbm.at[pl.ds(base + c * CHEL, CHEL)],
                              osems[c % NBUF]).wait()


def kernel(x, pos_table):
    x1 = x.reshape(B // 128, 128, SEQ, DIM // 8, 8)
    x1 = x1.transpose(2, 3, 0, 4, 1).reshape(N)
    o1 = _sc_add(x1, pos_table.reshape(SEQ * DIM))
    o5 = o1.reshape(SEQ, DIM // 8, B // 128, 8, 128)
    return o5.transpose(2, 4, 0, 1, 3).reshape(B, SEQ, DIM)
